# Initial kernel scaffold; baseline (speedup 1.0000x reference)
import jax
import jax.numpy as jnp
from jax.experimental import pallas as pl


def kernel(snip_feature, params):
    raise NotImplementedError("write your pallas kernel here")


# speedup vs baseline: 20.3032x; 20.3032x over previous
import functools

import jax
import jax.numpy as jnp
from jax import lax
from jax.experimental import pallas as pl
from jax.experimental.pallas import tpu as pltpu
from jax.experimental.pallas import tpu_sc as plsc

FEAT = 128
K = 6
GCN_G = 32
CONV_G = 4
B = 4
T = 2048
TILE = 256
NTILES = T // TILE
NEG = -3.0e38
F32 = jnp.float32


def _densify(w, groups):
    o, ig = w.shape[0], w.shape[1]
    rest = w.shape[2:]
    og = o // groups
    w5 = w.reshape((groups, og, ig) + rest)
    out = jnp.zeros((groups, og, groups, ig) + rest, w.dtype)
    out = out.at[jnp.arange(groups), :, jnp.arange(groups)].set(w5)
    return out.reshape((o, groups * ig) + rest)


def _taps(dense):
    return jnp.transpose(dense, (2, 1, 0))


def _dot(a, b):
    return jnp.dot(a, b, preferred_element_type=F32)


def _conv3(x, taps_ref, bias):
    acc = _dot(x, taps_ref[1]) + bias
    a0 = _dot(x, taps_ref[0])
    a2 = _dot(x, taps_ref[2])
    z = jnp.zeros((1, x.shape[1]), F32)
    acc = acc + jnp.concatenate([z, a0[:-1]], axis=0)
    acc = acc + jnp.concatenate([a2[1:], z], axis=0)
    return acc


def _backbone_body(x_ref, taps_ref, s_ref, b_ref, h_ref):
    x = x_ref[0]
    acc = _conv3(x, taps_ref, jnp.zeros((1, FEAT), F32))
    h_ref[0] = jnp.maximum(acc * s_ref[...] + b_ref[...], 0.0)


def _stage_a_body(h_ref, t1w_ref, t1b_ref, t2t_ref, t2b_ref, t3w_ref, t3b_ref,
                  wn_ref, wx_ref, s1b_ref, base_ref, yn_ref, yx_ref):
    h = h_ref[0]
    h1 = jnp.maximum(_dot(h, t1w_ref[...]) + t1b_ref[...], 0.0)
    h2 = jnp.maximum(_conv3(h1, t2t_ref, t2b_ref[...]), 0.0)
    base_ref[0] = _dot(h2, t3w_ref[...]) + t3b_ref[...] + h
    yn_ref[0] = _dot(h, wn_ref[...])
    yx_ref[0] = _dot(h, wx_ref[...]) + s1b_ref[...]


def _topk_body(h_ref, q_ref, idx_ref):
    b = pl.program_id(0)
    hf = h_ref[0]
    q = q_ref[0]
    inner2 = 2.0 * lax.dot_general(q, hf, (((1,), (1,)), ((), ())),
                                   preferred_element_type=F32)
    hsq = hf * hf
    xxf = lax.dot_general(jnp.ones((1, FEAT), F32), hsq,
                          (((1,), (1,)), ((), ())), preferred_element_type=F32)
    xxq = jnp.sum(q * q, axis=1, keepdims=True)
    d = inner2 - xxf - xxq
    cols = lax.broadcasted_iota(jnp.int32, (TILE, T), 1)
    outs = []
    for _ in range(K):
        m = jnp.max(d, axis=1, keepdims=True)
        idxj = jnp.min(jnp.where(d == m, cols, T), axis=1, keepdims=True)
        outs.append(idxj)
        d = jnp.where(cols == idxj, NEG, d)
    idx_ref[0] = jnp.concatenate(outs, axis=1) + b * T


def _stage_c_body(g_ref, yx_ref, base_ref, w2_ref, b2_ref, w3_ref, b3_ref,
                  out_ref):
    yx = yx_ref[0]
    acc = None
    for j in range(K):
        s1 = jnp.maximum(g_ref[0, j] + yx, 0.0)
        s2 = jnp.maximum(_dot(s1, w2_ref[...]) + b2_ref[...], 0.0)
        s3 = _dot(s2, w3_ref[...]) + b3_ref[...]
        acc = s3 if acc is None else jnp.maximum(acc, s3)
    out_ref[0] = jnp.maximum(base_ref[0] + acc, 0.0)


def _stage_c_final_body(g_ref, yx_ref, base_ref, resid_ref, w2_ref, b2_ref,
                        w3_ref, b3_ref, out_ref):
    yx = yx_ref[0]
    acc = None
    for j in range(K):
        s1 = jnp.maximum(g_ref[0, j] + yx, 0.0)
        s2 = jnp.maximum(_dot(s1, w2_ref[...]) + b2_ref[...], 0.0)
        s3 = _dot(s2, w3_ref[...]) + b3_ref[...]
        acc = s3 if acc is None else jnp.maximum(acc, s3)
    out_ref[0] = jnp.maximum(base_ref[0] + acc, 0.0) + resid_ref[0]


def _full_spec(shape):
    n = len(shape)
    return pl.BlockSpec(shape, lambda *_: (0,) * n)


def _bt_spec():
    return pl.BlockSpec((1, T, FEAT), lambda b, *_: (b, 0, 0))


_TC_PARAMS = pltpu.CompilerParams(vmem_limit_bytes=100 * 2**20)


def _backbone(x_t, taps, s, b):
    return pl.pallas_call(
        _backbone_body,
        grid=(B,),
        in_specs=[_bt_spec(), _full_spec(taps.shape), _full_spec(s.shape),
                  _full_spec(b.shape)],
        out_specs=_bt_spec(),
        out_shape=jax.ShapeDtypeStruct((B, T, FEAT), F32),
        compiler_params=_TC_PARAMS,
    )(x_t, taps, s, b)


def _stage_a(h, t1w, t1b, t2t, t2b, t3w, t3b, wn, wx, s1b):
    ws = [t1w, t1b, t2t, t2b, t3w, t3b, wn, wx, s1b]
    out_sd = jax.ShapeDtypeStruct((B, T, FEAT), F32)
    return pl.pallas_call(
        _stage_a_body,
        grid=(B,),
        in_specs=[_bt_spec()] + [_full_spec(w.shape) for w in ws],
        out_specs=[_bt_spec(), _bt_spec(), _bt_spec()],
        out_shape=[out_sd, out_sd, out_sd],
        compiler_params=_TC_PARAMS,
    )(h, *ws)


def _topk(h):
    return pl.pallas_call(
        _topk_body,
        grid=(B, NTILES),
        in_specs=[
            pl.BlockSpec((1, T, FEAT), lambda b, i: (b, 0, 0)),
            pl.BlockSpec((1, TILE, FEAT), lambda b, i: (b, i, 0)),
        ],
        out_specs=pl.BlockSpec((1, TILE, K), lambda b, i: (b, i, 0)),
        out_shape=jax.ShapeDtypeStruct((B, T, K), jnp.int32),
        compiler_params=_TC_PARAMS,
    )(h, h)


def _stage_c(body, gat, yx, base, extra, w2, b2, w3, b3):
    tile_spec = pl.BlockSpec((1, TILE, FEAT), lambda b, i: (b, i, 0))
    ws = [w2, b2, w3, b3]
    in_specs = [pl.BlockSpec((1, K, TILE, FEAT), lambda b, i: (b, 0, i, 0)),
                tile_spec, tile_spec]
    args = [gat, yx, base]
    if extra is not None:
        in_specs.append(tile_spec)
        args.append(extra)
    in_specs += [_full_spec(w.shape) for w in ws]
    args += ws
    return pl.pallas_call(
        body,
        grid=(B, NTILES),
        in_specs=in_specs,
        out_specs=tile_spec,
        out_shape=jax.ShapeDtypeStruct((B, T, FEAT), F32),
        compiler_params=_TC_PARAMS,
    )(*args)


_NW = 32
_ROWS = B * K * T
_PER_W = _ROWS // _NW
_CH = 128
_NCH = _PER_W // _CH


def _sc_gather_impl(table, gidx):
    mesh = plsc.VectorSubcoreMesh(core_axis_name="c", subcore_axis_name="s")

    @functools.partial(
        pl.kernel,
        out_type=jax.ShapeDtypeStruct((_ROWS, FEAT), F32),
        mesh=mesh,
        scratch_types=[
            pltpu.VMEM((_CH,), jnp.int32),
            pltpu.VMEM((_CH, FEAT), F32),
            pltpu.SemaphoreType.DMA,
        ],
    )
    def gather_k(table_hbm, idx_hbm, out_hbm, idx_v, rows_v, sem):
        wid = lax.axis_index("s") * 2 + lax.axis_index("c")
        base = wid * _PER_W
        for c in range(_NCH):
            off = base + c * _CH
            pltpu.sync_copy(idx_hbm.at[pl.ds(off, _CH)], idx_v)
            pltpu.async_copy(table_hbm.at[idx_v], rows_v, sem).wait()
            pltpu.sync_copy(rows_v, out_hbm.at[pl.ds(off, _CH)])

    return gather_k(table, gidx)


def _gather_rows(table, gidx):
    return _sc_gather_impl(table, gidx)


def _row(v):
    return v.reshape(1, FEAT)


def kernel(snip_feature, params):
    p = params
    x_t = jnp.transpose(snip_feature, (0, 2, 1))

    bb_taps = _taps(_densify(p['bb_w'], CONV_G))
    inv = 1.0 / jnp.sqrt(jnp.float32(1.0 + 1e-05))
    eff_s = _row(p['bn_g'] * inv)
    eff_b = _row(p['bb_b'] * p['bn_g'] * inv + p['bn_b'])
    h = _backbone(x_t, bb_taps, eff_s, eff_b)

    for g, final in (('g1', False), ('g2', True)):
        t1w = jnp.transpose(p[g + '_t1_w'][:, :, 0])
        t2t = _taps(_densify(p[g + '_t2_w'], GCN_G))
        t3w = jnp.transpose(p[g + '_t3_w'][:, :, 0])
        s1w = p[g + '_s1_w'][:, :, 0, 0]
        wn = jnp.transpose(s1w[:, :FEAT])
        wx = jnp.transpose(s1w[:, FEAT:])
        w2 = jnp.transpose(_densify(p[g + '_s2_w'][:, :, 0, 0], GCN_G))
        w3 = jnp.transpose(p[g + '_s3_w'][:, :, 0, 0])

        base, yn, yx = _stage_a(
            h, t1w, _row(p[g + '_t1_b']), t2t, _row(p[g + '_t2_b']),
            t3w, _row(p[g + '_t3_b']), wn, wx, _row(p[g + '_s1_b']))
        idx = _topk(h)
        gidx = jnp.transpose(idx, (0, 2, 1)).reshape(-1)
        gat = _gather_rows(yn.reshape(B * T, FEAT), gidx)
        gat = gat.reshape(B, K, T, FEAT)
        if final:
            out_t = _stage_c(_stage_c_final_body, gat, yx, base, x_t,
                             w2, _row(p[g + '_s2_b']),
                             w3, _row(p[g + '_s3_b']))
        else:
            h = _stage_c(_stage_c_body, gat, yx, base, None,
                         w2, _row(p[g + '_s2_b']),
                         w3, _row(p[g + '_s3_b']))

    return jnp.transpose(out_t, (0, 2, 1))


# speedup vs baseline: 20.7271x; 1.0209x over previous
import functools

import jax
import jax.numpy as jnp
from jax import lax
from jax.experimental import pallas as pl
from jax.experimental.pallas import tpu as pltpu
from jax.experimental.pallas import tpu_sc as plsc

FEAT = 128
K = 6
GCN_G = 32
CONV_G = 4
B = 4
T = 2048
TILE = 256
NTILES = T // TILE
NEG = -3.0e38
F32 = jnp.float32


def _densify(w, groups):
    o, ig = w.shape[0], w.shape[1]
    rest = w.shape[2:]
    og = o // groups
    w5 = w.reshape((groups, og, ig) + rest)
    out = jnp.zeros((groups, og, groups, ig) + rest, w.dtype)
    out = out.at[jnp.arange(groups), :, jnp.arange(groups)].set(w5)
    return out.reshape((o, groups * ig) + rest)


def _taps(dense):
    return jnp.transpose(dense, (2, 1, 0))


def _dot(a, b):
    return jnp.dot(a, b, preferred_element_type=F32)


def _conv3(x, taps_ref, bias):
    acc = _dot(x, taps_ref[1]) + bias
    a0 = _dot(x, taps_ref[0])
    a2 = _dot(x, taps_ref[2])
    z = jnp.zeros((1, x.shape[1]), F32)
    acc = acc + jnp.concatenate([z, a0[:-1]], axis=0)
    acc = acc + jnp.concatenate([a2[1:], z], axis=0)
    return acc


def _backbone_body(x_ref, taps_ref, s_ref, b_ref, h_ref):
    x = x_ref[0]
    acc = _conv3(x, taps_ref, jnp.zeros((1, FEAT), F32))
    h_ref[0] = jnp.maximum(acc * s_ref[...] + b_ref[...], 0.0)


def _yn_body(h_ref, wn_ref, yn_ref):
    yn_ref[0] = _dot(h_ref[0], wn_ref[...])


def _stage_a_body(h_ref, t1w_ref, t1b_ref, t2t_ref, t2b_ref, t3w_ref, t3b_ref,
                  wx_ref, s1b_ref, base_ref, yx_ref):
    h = h_ref[0]
    h1 = jnp.maximum(_dot(h, t1w_ref[...]) + t1b_ref[...], 0.0)
    h2 = jnp.maximum(_conv3(h1, t2t_ref, t2b_ref[...]), 0.0)
    base_ref[0] = _dot(h2, t3w_ref[...]) + t3b_ref[...] + h
    yx_ref[0] = _dot(h, wx_ref[...]) + s1b_ref[...]


def _topk_body(b0, h_ref, q_ref, idx_ref):
    hf = h_ref[0]
    q = q_ref[0]
    inner2 = 2.0 * lax.dot_general(q, hf, (((1,), (1,)), ((), ())),
                                   preferred_element_type=F32)
    hsq = hf * hf
    xxf = lax.dot_general(jnp.ones((1, FEAT), F32), hsq,
                          (((1,), (1,)), ((), ())), preferred_element_type=F32)
    d = inner2 - xxf
    cols = lax.broadcasted_iota(jnp.int32, (TILE, T), 1)
    outs = []
    for _ in range(K):
        m = jnp.max(d, axis=1, keepdims=True)
        sel = d == m
        idxj = jnp.min(jnp.where(sel, cols, T), axis=1, keepdims=True)
        outs.append(idxj)
        d = jnp.where(sel, NEG, d)
    idx_ref[0] = jnp.concatenate(outs, axis=1) + b0 * T


def _stage_c_body(g_ref, yx_ref, base_ref, w2_ref, b2_ref, w3_ref, b3_ref,
                  out_ref):
    yx = yx_ref[0]
    acc = None
    for j in range(K):
        s1 = jnp.maximum(g_ref[0, j] + yx, 0.0)
        s2 = jnp.maximum(_dot(s1, w2_ref[...]) + b2_ref[...], 0.0)
        s3 = _dot(s2, w3_ref[...]) + b3_ref[...]
        acc = s3 if acc is None else jnp.maximum(acc, s3)
    out_ref[0] = jnp.maximum(base_ref[0] + acc, 0.0)


def _stage_c_final_body(g_ref, yx_ref, base_ref, resid_ref, w2_ref, b2_ref,
                        w3_ref, b3_ref, out_ref):
    yx = yx_ref[0]
    acc = None
    for j in range(K):
        s1 = jnp.maximum(g_ref[0, j] + yx, 0.0)
        s2 = jnp.maximum(_dot(s1, w2_ref[...]) + b2_ref[...], 0.0)
        s3 = _dot(s2, w3_ref[...]) + b3_ref[...]
        acc = s3 if acc is None else jnp.maximum(acc, s3)
    out_ref[0] = jnp.maximum(base_ref[0] + acc, 0.0) + resid_ref[0]


def _full_spec(shape):
    n = len(shape)
    return pl.BlockSpec(shape, lambda *_: (0,) * n)


def _bt_spec():
    return pl.BlockSpec((1, T, FEAT), lambda b, *_: (b, 0, 0))


_TC_PARAMS = pltpu.CompilerParams(vmem_limit_bytes=100 * 2**20)
_BT_SD = jax.ShapeDtypeStruct((B, T, FEAT), F32)


def _backbone(x_t, taps, s, b):
    return pl.pallas_call(
        _backbone_body,
        grid=(B,),
        in_specs=[_bt_spec(), _full_spec(taps.shape), _full_spec(s.shape),
                  _full_spec(b.shape)],
        out_specs=_bt_spec(),
        out_shape=_BT_SD,
        compiler_params=_TC_PARAMS,
    )(x_t, taps, s, b)


def _yn_call(h, wn):
    return pl.pallas_call(
        _yn_body,
        grid=(B,),
        in_specs=[_bt_spec(), _full_spec(wn.shape)],
        out_specs=_bt_spec(),
        out_shape=_BT_SD,
        compiler_params=_TC_PARAMS,
    )(h, wn)


def _stage_a(h, *ws):
    return pl.pallas_call(
        _stage_a_body,
        grid=(B,),
        in_specs=[_bt_spec()] + [_full_spec(w.shape) for w in ws],
        out_specs=[_bt_spec(), _bt_spec()],
        out_shape=[_BT_SD, _BT_SD],
        compiler_params=_TC_PARAMS,
    )(h, *ws)


def _topk_part(h, b0):
    return pl.pallas_call(
        functools.partial(_topk_body, b0),
        grid=(NTILES,),
        in_specs=[
            pl.BlockSpec((1, T, FEAT), lambda i: (b0, 0, 0)),
            pl.BlockSpec((1, TILE, FEAT), lambda i: (b0, i, 0)),
        ],
        out_specs=pl.BlockSpec((1, TILE, K), lambda i: (0, i, 0)),
        out_shape=jax.ShapeDtypeStruct((1, T, K), jnp.int32),
        compiler_params=_TC_PARAMS,
    )(h, h)


def _stage_c_part(body, b0, gat, yx, base, extra, ws):
    def bslice(i):
        return (b0, i, 0)

    tile_spec = pl.BlockSpec((1, TILE, FEAT), bslice)
    in_specs = [pl.BlockSpec((1, K, TILE, FEAT), lambda i: (0, 0, i, 0)),
                tile_spec, tile_spec]
    args = [gat, yx, base]
    if extra is not None:
        in_specs.append(tile_spec)
        args.append(extra)
    in_specs += [_full_spec(w.shape) for w in ws]
    args += list(ws)
    return pl.pallas_call(
        body,
        grid=(NTILES,),
        in_specs=in_specs,
        out_specs=pl.BlockSpec((1, TILE, FEAT), lambda i: (0, i, 0)),
        out_shape=jax.ShapeDtypeStruct((1, T, FEAT), F32),
        compiler_params=_TC_PARAMS,
    )(*args)


_NW = 32
_CH = 128


def _sc_gather_impl(table, gidx):
    rows = gidx.shape[0]
    per_w = rows // _NW
    nch = per_w // _CH
    mesh = plsc.VectorSubcoreMesh(core_axis_name="c", subcore_axis_name="s")

    @functools.partial(
        pl.kernel,
        out_type=jax.ShapeDtypeStruct((rows, FEAT), F32),
        mesh=mesh,
        scratch_types=[
            pltpu.VMEM((_CH,), jnp.int32),
            pltpu.VMEM((_CH, FEAT), F32),
            pltpu.SemaphoreType.DMA,
        ],
    )
    def gather_k(table_hbm, idx_hbm, out_hbm, idx_v, rows_v, sem):
        wid = lax.axis_index("s") * 2 + lax.axis_index("c")
        base = wid * per_w
        for c in range(nch):
            off = base + c * _CH
            pltpu.sync_copy(idx_hbm.at[pl.ds(off, _CH)], idx_v)
            pltpu.async_copy(table_hbm.at[idx_v], rows_v, sem).wait()
            pltpu.sync_copy(rows_v, out_hbm.at[pl.ds(off, _CH)])

    return gather_k(table, gidx)


def _gather_rows(table, gidx):
    return _sc_gather_impl(table, gidx)


def _row(v):
    return v.reshape(1, FEAT)


def kernel(snip_feature, params):
    p = params
    x_t = jnp.transpose(snip_feature, (0, 2, 1))

    bb_taps = _taps(_densify(p['bb_w'], CONV_G))
    inv = 1.0 / jnp.sqrt(jnp.float32(1.0 + 1e-05))
    eff_s = _row(p['bn_g'] * inv)
    eff_b = _row(p['bb_b'] * p['bn_g'] * inv + p['bn_b'])
    h = _backbone(x_t, bb_taps, eff_s, eff_b)

    for g, final in (('g1', False), ('g2', True)):
        t1w = jnp.transpose(p[g + '_t1_w'][:, :, 0])
        t2t = _taps(_densify(p[g + '_t2_w'], GCN_G))
        t3w = jnp.transpose(p[g + '_t3_w'][:, :, 0])
        s1w = p[g + '_s1_w'][:, :, 0, 0]
        wn = jnp.transpose(s1w[:, :FEAT])
        wx = jnp.transpose(s1w[:, FEAT:])
        w2 = jnp.transpose(_densify(p[g + '_s2_w'][:, :, 0, 0], GCN_G))
        w3 = jnp.transpose(p[g + '_s3_w'][:, :, 0, 0])

        yn = _yn_call(h, wn)
        tbl = yn.reshape(B * T, FEAT)
        gats = []
        for b0 in range(B):
            idxp = _topk_part(h, b0)
            gidx = jnp.transpose(idxp, (0, 2, 1)).reshape(-1)
            gats.append(_gather_rows(tbl, gidx).reshape(1, K, T, FEAT))
        base, yx = _stage_a(
            h, t1w, _row(p[g + '_t1_b']), t2t, _row(p[g + '_t2_b']),
            t3w, _row(p[g + '_t3_b']), wx, _row(p[g + '_s1_b']))
        cws = (w2, _row(p[g + '_s2_b']), w3, _row(p[g + '_s3_b']))
        if final:
            outs = [_stage_c_part(_stage_c_final_body, b0, gats[b0], yx, base,
                                  x_t, cws) for b0 in range(B)]
            out_t = jnp.concatenate(outs, axis=0)
        else:
            outs = [_stage_c_part(_stage_c_body, b0, gats[b0], yx, base,
                                  None, cws) for b0 in range(B)]
            h = jnp.concatenate(outs, axis=0)

    return jnp.transpose(out_t, (0, 2, 1))


# speedup vs baseline: 23.4352x; 1.1307x over previous
import functools

import jax
import jax.numpy as jnp
from jax import lax
from jax.experimental import pallas as pl
from jax.experimental.pallas import tpu as pltpu
from jax.experimental.pallas import tpu_sc as plsc

FEAT = 128
K = 6
GCN_G = 32
CONV_G = 4
B = 4
T = 2048
TILE = 256
NTILES = T // TILE
NEG = -3.0e38
F32 = jnp.float32


def _densify(w, groups):
    o, ig = w.shape[0], w.shape[1]
    rest = w.shape[2:]
    og = o // groups
    w5 = w.reshape((groups, og, ig) + rest)
    out = jnp.zeros((groups, og, groups, ig) + rest, w.dtype)
    out = out.at[jnp.arange(groups), :, jnp.arange(groups)].set(w5)
    return out.reshape((o, groups * ig) + rest)


def _taps(dense):
    return jnp.transpose(dense, (2, 1, 0))


def _dot(a, b):
    return jnp.dot(a, b, preferred_element_type=F32)


def _conv3(x, taps_ref, bias):
    acc = _dot(x, taps_ref[1]) + bias
    a0 = _dot(x, taps_ref[0])
    a2 = _dot(x, taps_ref[2])
    z = jnp.zeros((1, x.shape[1]), F32)
    acc = acc + jnp.concatenate([z, a0[:-1]], axis=0)
    acc = acc + jnp.concatenate([a2[1:], z], axis=0)
    return acc


def _backbone_body(x_ref, taps_ref, s_ref, b_ref, h_ref):
    x = x_ref[0]
    acc = _conv3(x, taps_ref, jnp.zeros((1, FEAT), F32))
    h_ref[0] = jnp.maximum(acc * s_ref[...] + b_ref[...], 0.0)


def _yn_body(h_ref, wn_ref, yn_ref):
    yn_ref[0] = _dot(h_ref[0], wn_ref[...])


def _stage_a_body(h_ref, t1w_ref, t1b_ref, t2t_ref, t2b_ref, t3w_ref, t3b_ref,
                  wx_ref, s1b_ref, base_ref, yx_ref):
    h = h_ref[0]
    h1 = jnp.maximum(_dot(h, t1w_ref[...]) + t1b_ref[...], 0.0)
    h2 = jnp.maximum(_conv3(h1, t2t_ref, t2b_ref[...]), 0.0)
    base_ref[0] = _dot(h2, t3w_ref[...]) + t3b_ref[...] + h
    yx_ref[0] = _dot(h, wx_ref[...]) + s1b_ref[...]


def _topk_body(h_ref, q_ref, idx_ref):
    b0 = pl.program_id(0)
    hf = h_ref[0]
    q = q_ref[0]
    inner2 = 2.0 * lax.dot_general(q, hf, (((1,), (1,)), ((), ())),
                                   preferred_element_type=F32)
    hsq = hf * hf
    xxf = lax.dot_general(jnp.ones((1, FEAT), F32), hsq,
                          (((1,), (1,)), ((), ())), preferred_element_type=F32)
    d = inner2 - xxf
    qt = q.shape[0]
    i = pl.program_id(1)
    cols = lax.broadcasted_iota(jnp.int32, (qt, T), 1)
    rows = i * qt + lax.broadcasted_iota(jnp.int32, (qt, 1), 0)
    outs = [rows]
    d = jnp.where(cols == rows, NEG, d)
    for _ in range(K - 1):
        m = jnp.max(d, axis=1, keepdims=True)
        sel = d == m
        idxj = jnp.min(jnp.where(sel, cols, T), axis=1, keepdims=True)
        outs.append(idxj)
        d = jnp.where(sel, NEG, d)
    idx_ref[0] = jnp.concatenate(outs, axis=1) + b0 * T


def _stage_c_body(g_ref, yx_ref, base_ref, w2_ref, b2_ref, w3_ref, b3_ref,
                  out_ref):
    yx = yx_ref[0]
    acc = None
    for j in range(K):
        s1 = jnp.maximum(g_ref[0, j] + yx, 0.0)
        s2 = jnp.maximum(_dot(s1, w2_ref[...]) + b2_ref[...], 0.0)
        s3 = _dot(s2, w3_ref[...]) + b3_ref[...]
        acc = s3 if acc is None else jnp.maximum(acc, s3)
    out_ref[0] = jnp.maximum(base_ref[0] + acc, 0.0)


def _stage_c_final_body(g_ref, yx_ref, base_ref, resid_ref, w2_ref, b2_ref,
                        w3_ref, b3_ref, out_ref):
    yx = yx_ref[0]
    acc = None
    for j in range(K):
        s1 = jnp.maximum(g_ref[0, j] + yx, 0.0)
        s2 = jnp.maximum(_dot(s1, w2_ref[...]) + b2_ref[...], 0.0)
        s3 = _dot(s2, w3_ref[...]) + b3_ref[...]
        acc = s3 if acc is None else jnp.maximum(acc, s3)
    out_ref[0] = jnp.maximum(base_ref[0] + acc, 0.0) + resid_ref[0]


def _full_spec(shape):
    n = len(shape)
    return pl.BlockSpec(shape, lambda *_: (0,) * n)


def _bt_spec():
    return pl.BlockSpec((1, T, FEAT), lambda b, *_: (b, 0, 0))


_TC_PARAMS = pltpu.CompilerParams(vmem_limit_bytes=100 * 2**20)
_BT_SD = jax.ShapeDtypeStruct((B, T, FEAT), F32)


def _backbone(x_t, taps, s, b):
    return pl.pallas_call(
        _backbone_body,
        grid=(B,),
        in_specs=[_bt_spec(), _full_spec(taps.shape), _full_spec(s.shape),
                  _full_spec(b.shape)],
        out_specs=_bt_spec(),
        out_shape=_BT_SD,
        compiler_params=_TC_PARAMS,
    )(x_t, taps, s, b)


def _yn_call(h, wn):
    return pl.pallas_call(
        _yn_body,
        grid=(B,),
        in_specs=[_bt_spec(), _full_spec(wn.shape)],
        out_specs=_bt_spec(),
        out_shape=_BT_SD,
        compiler_params=_TC_PARAMS,
    )(h, wn)


def _stage_a(h, *ws):
    return pl.pallas_call(
        _stage_a_body,
        grid=(B,),
        in_specs=[_bt_spec()] + [_full_spec(w.shape) for w in ws],
        out_specs=[_bt_spec(), _bt_spec()],
        out_shape=[_BT_SD, _BT_SD],
        compiler_params=_TC_PARAMS,
    )(h, *ws)


_QT = 512


def _topk(h):
    return pl.pallas_call(
        _topk_body,
        grid=(B, T // _QT),
        in_specs=[
            pl.BlockSpec((1, T, FEAT), lambda b, i: (b, 0, 0)),
            pl.BlockSpec((1, _QT, FEAT), lambda b, i: (b, i, 0)),
        ],
        out_specs=pl.BlockSpec((1, _QT, K), lambda b, i: (b, i, 0)),
        out_shape=jax.ShapeDtypeStruct((B, T, K), jnp.int32),
        compiler_params=_TC_PARAMS,
    )(h, h)


def _stage_c(body, gat, yx, base, extra, ws):
    tile_spec = pl.BlockSpec((1, TILE, FEAT), lambda b, i: (b, i, 0))
    in_specs = [pl.BlockSpec((1, K, TILE, FEAT), lambda b, i: (b, 0, i, 0)),
                tile_spec, tile_spec]
    args = [gat, yx, base]
    if extra is not None:
        in_specs.append(tile_spec)
        args.append(extra)
    in_specs += [_full_spec(w.shape) for w in ws]
    args += list(ws)
    return pl.pallas_call(
        body,
        grid=(B, NTILES),
        in_specs=in_specs,
        out_specs=tile_spec,
        out_shape=_BT_SD,
        compiler_params=_TC_PARAMS,
    )(*args)


_NW = 32
_CH = 128


def _sc_gather_impl(table, gidx):
    rows = gidx.shape[0]
    per_w = rows // _NW
    nch = per_w // _CH
    mesh = plsc.VectorSubcoreMesh(core_axis_name="c", subcore_axis_name="s")

    @functools.partial(
        pl.kernel,
        out_type=jax.ShapeDtypeStruct((rows, FEAT), F32),
        mesh=mesh,
        scratch_types=[
            pltpu.VMEM((per_w,), jnp.int32),
            pltpu.VMEM((_CH, FEAT), F32),
            pltpu.VMEM((_CH, FEAT), F32),
            pltpu.SemaphoreType.DMA,
            pltpu.SemaphoreType.DMA,
            pltpu.SemaphoreType.DMA,
            pltpu.SemaphoreType.DMA,
        ],
    )
    def gather_k(table_hbm, idx_hbm, out_hbm, idx_v, buf0, buf1,
                 g0, g1, s0, s1):
        wid = lax.axis_index("s") * 2 + lax.axis_index("c")
        base = wid * per_w
        bufs = (buf0, buf1)
        gsems = (g0, g1)
        ssems = (s0, s1)
        pltpu.sync_copy(idx_hbm.at[pl.ds(base, per_w)], idx_v)
        gathers = [None, None]
        stores = [None, None]
        gathers[0] = pltpu.async_copy(
            table_hbm.at[idx_v.at[pl.ds(0, _CH)]], bufs[0], g0)
        for c in range(nch):
            cur = c % 2
            nxt = (c + 1) % 2
            if c + 1 < nch:
                if stores[nxt] is not None:
                    stores[nxt].wait()
                gathers[nxt] = pltpu.async_copy(
                    table_hbm.at[idx_v.at[pl.ds((c + 1) * _CH, _CH)]],
                    bufs[nxt], gsems[nxt])
            gathers[cur].wait()
            stores[cur] = pltpu.async_copy(
                bufs[cur], out_hbm.at[pl.ds(base + c * _CH, _CH)], ssems[cur])
        stores[(nch - 1) % 2].wait()
        if nch > 1:
            stores[nch % 2].wait()

    return gather_k(table, gidx)


def _gather_rows(table, gidx):
    return _sc_gather_impl(table, gidx)


def _row(v):
    return v.reshape(1, FEAT)


def kernel(snip_feature, params):
    p = params
    x_t = jnp.transpose(snip_feature, (0, 2, 1))

    bb_taps = _taps(_densify(p['bb_w'], CONV_G))
    inv = 1.0 / jnp.sqrt(jnp.float32(1.0 + 1e-05))
    eff_s = _row(p['bn_g'] * inv)
    eff_b = _row(p['bb_b'] * p['bn_g'] * inv + p['bn_b'])
    h = _backbone(x_t, bb_taps, eff_s, eff_b)

    for g, final in (('g1', False), ('g2', True)):
        t1w = jnp.transpose(p[g + '_t1_w'][:, :, 0])
        t2t = _taps(_densify(p[g + '_t2_w'], GCN_G))
        t3w = jnp.transpose(p[g + '_t3_w'][:, :, 0])
        s1w = p[g + '_s1_w'][:, :, 0, 0]
        wn = jnp.transpose(s1w[:, :FEAT])
        wx = jnp.transpose(s1w[:, FEAT:])
        w2 = jnp.transpose(_densify(p[g + '_s2_w'][:, :, 0, 0], GCN_G))
        w3 = jnp.transpose(p[g + '_s3_w'][:, :, 0, 0])

        yn = _yn_call(h, wn)
        idx = _topk(h)
        gidx = jnp.transpose(idx, (0, 2, 1)).reshape(-1)
        gat = _gather_rows(yn.reshape(B * T, FEAT), gidx)
        gat = gat.reshape(B, K, T, FEAT)
        base, yx = _stage_a(
            h, t1w, _row(p[g + '_t1_b']), t2t, _row(p[g + '_t2_b']),
            t3w, _row(p[g + '_t3_b']), wx, _row(p[g + '_s1_b']))
        cws = (w2, _row(p[g + '_s2_b']), w3, _row(p[g + '_s3_b']))
        if final:
            out_t = _stage_c(_stage_c_final_body, gat, yx, base, x_t, cws)
        else:
            h = _stage_c(_stage_c_body, gat, yx, base, None, cws)

    return jnp.transpose(out_t, (0, 2, 1))
